# baseline (speedup 1.0000x reference)
import functools

import jax
import jax.numpy as jnp
from jax import lax
from jax.experimental import pallas as pl
from jax.experimental.pallas import tpu as pltpu
from jax.experimental.pallas import tpu_sc as plsc


@functools.lru_cache(maxsize=None)
def _make_gather(V2, D2, B):
    D = D2 // 2
    info = plsc.get_sparse_core_info()
    NC, NS = info.num_cores, info.num_subcores
    NW = NC * NS
    assert B % (8 * NW) == 0, (B, NW)
    b_per_w = B // NW
    C = 256
    while b_per_w % (2 * C):
        C //= 2
    n_chunks = b_per_w // C
    mesh = plsc.VectorSubcoreMesh(core_axis_name="c", subcore_axis_name="s")

    @functools.partial(
        pl.kernel,
        mesh=mesh,
        out_type=jax.ShapeDtypeStruct((B // 2, D2), jnp.float32),
        scratch_types=[
            pltpu.VMEM((b_per_w,), jnp.int32),
            pltpu.VMEM((b_per_w,), jnp.int32),
            pltpu.VMEM((C, D2), jnp.float32),
            pltpu.VMEM((C, D2), jnp.float32),
            pltpu.VMEM((C // 2, D2), jnp.float32),
            pltpu.VMEM((C // 2, D2), jnp.float32),
            pltpu.SemaphoreType.DMA,
            pltpu.SemaphoreType.DMA,
            pltpu.SemaphoreType.DMA,
            pltpu.SemaphoreType.DMA,
        ],
    )
    def k(idx_hbm, table_hbm, out_hbm, idx_v, idx2_v, p0, p1, o0, o1,
          gs0, gs1, ws0, ws1):
        pairs_b = (p0, p1)
        out_b = (o0, o1)
        gs = (gs0, gs1)
        ws = (ws0, ws1)
        wid = lax.axis_index("s") * NC + lax.axis_index("c")
        base = wid * b_per_w

        pltpu.sync_copy(idx_hbm.at[pl.ds(base, b_per_w)], idx_v)

        def prep(i, carry):
            v = idx_v[pl.ds(i * 16, 16)]
            idx2_v[pl.ds(i * 16, 16)] = lax.shift_right_logical(v, 1)
            return carry

        lax.fori_loop(0, b_per_w // 16, prep, 0)

        def start_gather(g, b):
            pltpu.async_copy(
                table_hbm.at[idx2_v.at[pl.ds(g * C, C)]], pairs_b[b], gs[b]
            )

        def wait_gather(b):
            pltpu.make_async_copy(
                table_hbm.at[idx2_v.at[pl.ds(0, C)]], pairs_b[b], gs[b]
            ).wait()

        def start_write(g, b):
            off = pl.multiple_of((base + g * C) // 2, 8)
            pltpu.async_copy(out_b[b], out_hbm.at[pl.ds(off, C // 2)], ws[b])

        def wait_write(b):
            pltpu.make_async_copy(
                out_b[b], out_hbm.at[pl.ds(pl.multiple_of(base // 2, 8), C // 2)],
                ws[b],
            ).wait()

        def compact(g, b):
            pairs = pairs_b[b]
            out = out_b[b]

            def blk16(q, carry):
                r0 = 16 * q
                par = (idx_v[pl.ds(g * C + r0, 16)] & 1) * D
                for j in range(16):
                    p64 = par[j]
                    for kblk in range(D // 16):
                        out[8 * q + j // 2, pl.ds((j % 2) * D + kblk * 16, 16)] = (
                            pairs[r0 + j, pl.ds(p64 + kblk * 16, 16)]
                        )
                return carry

            lax.fori_loop(0, C // 16, blk16, 0)

        start_gather(0, 0)
        start_gather(1, 1)

        def chunk(p, carry):
            g0 = 2 * p
            for b in range(2):
                g = g0 + b
                wait_gather(b)

                @pl.when(g >= 2)
                def _():
                    wait_write(b)

                compact(g, b)
                start_write(g, b)

                @pl.when(g + 2 < n_chunks)
                def _():
                    start_gather(g + 2, b)
            return carry

        lax.fori_loop(0, n_chunks // 2, chunk, 0)
        wait_write(0)
        wait_write(1)

    return k


def kernel(index, cluster_index):
    B_rows, F = index.shape
    V, D = cluster_index.shape
    B = B_rows * F
    idx_flat = index.reshape(B)
    table2 = cluster_index.reshape(V // 2, 2 * D)
    out = _make_gather(V // 2, 2 * D, B)(idx_flat, table2)
    return out.reshape(B_rows, F, D)
